# Initial kernel scaffold; baseline (speedup 1.0000x reference)
import jax
import jax.numpy as jnp
from jax.experimental import pallas as pl


def kernel(x, seq_len, emb_table, norm_weight):
    raise NotImplementedError("write your pallas kernel here")


# speedup vs baseline: 2.0326x; 2.0326x over previous
import functools

import jax
import jax.numpy as jnp
from jax.experimental import pallas as pl
from jax.experimental.pallas import tpu as pltpu

DIM = 4096
EPS = 1e-05
SEQ_TILE = 256


def _fused_kernel(seq_len_ref, x_ref, emb_ref, w_ref, out_ref):
    s = pl.program_id(0)
    seq_len = seq_len_ref[0]
    rows = jax.lax.broadcasted_iota(jnp.int32, (SEQ_TILE, 1), 0) + s * SEQ_TILE
    emb = jnp.where(rows < seq_len, emb_ref[...], 0.0)
    h = x_ref[0] + emb
    var = jnp.mean(h * h, axis=-1, keepdims=True)
    out_ref[0] = h * jax.lax.rsqrt(var + EPS) * w_ref[0]


@functools.partial(jax.jit, static_argnames=())
def kernel(x, seq_len, emb_table, norm_weight):
    batch, seq, dim = x.shape
    assert dim == DIM and seq % SEQ_TILE == 0
    seq_tiles = seq // SEQ_TILE
    seq_len_arr = jnp.asarray(seq_len, dtype=jnp.int32).reshape((1,))
    w2d = norm_weight.reshape(1, dim)

    grid_spec = pltpu.PrefetchScalarGridSpec(
        num_scalar_prefetch=1,
        grid=(seq_tiles, batch),
        in_specs=[
            pl.BlockSpec((1, SEQ_TILE, dim), lambda s, b, *_: (b, s, 0)),
            pl.BlockSpec((SEQ_TILE, dim), lambda s, b, *_: (s, 0)),
            pl.BlockSpec((1, dim), lambda s, b, *_: (0, 0)),
        ],
        out_specs=pl.BlockSpec((1, SEQ_TILE, dim), lambda s, b, *_: (b, s, 0)),
    )
    return pl.pallas_call(
        _fused_kernel,
        grid_spec=grid_spec,
        out_shape=jax.ShapeDtypeStruct(x.shape, x.dtype),
    )(seq_len_arr, x, emb_table, w2d)


# speedup vs baseline: 2.1023x; 1.0343x over previous
import functools

import jax
import jax.numpy as jnp
from jax.experimental import pallas as pl
from jax.experimental.pallas import tpu as pltpu

DIM = 4096
EPS = 1e-05
SEQ_TILE = 512


def _fused_kernel(seq_len_ref, x_ref, emb_ref, w_ref, out_ref):
    s = pl.program_id(0)
    seq_len = seq_len_ref[0]
    rows = jax.lax.broadcasted_iota(jnp.int32, (SEQ_TILE, 1), 0) + s * SEQ_TILE
    emb = jnp.where(rows < seq_len, emb_ref[...], 0.0)
    h = x_ref[0] + emb
    var = jnp.mean(h * h, axis=-1, keepdims=True)
    out_ref[0] = h * jax.lax.rsqrt(var + EPS) * w_ref[0]


@functools.partial(jax.jit, static_argnames=())
def kernel(x, seq_len, emb_table, norm_weight):
    batch, seq, dim = x.shape
    assert dim == DIM and seq % SEQ_TILE == 0
    seq_tiles = seq // SEQ_TILE
    seq_len_arr = jnp.asarray(seq_len, dtype=jnp.int32).reshape((1,))
    w2d = norm_weight.reshape(1, dim)

    grid_spec = pltpu.PrefetchScalarGridSpec(
        num_scalar_prefetch=1,
        grid=(seq_tiles, batch),
        in_specs=[
            pl.BlockSpec((1, SEQ_TILE, dim), lambda s, b, *_: (b, s, 0)),
            pl.BlockSpec((SEQ_TILE, dim), lambda s, b, *_: (s, 0)),
            pl.BlockSpec((1, dim), lambda s, b, *_: (0, 0)),
        ],
        out_specs=pl.BlockSpec((1, SEQ_TILE, dim), lambda s, b, *_: (b, s, 0)),
    )
    return pl.pallas_call(
        _fused_kernel,
        grid_spec=grid_spec,
        out_shape=jax.ShapeDtypeStruct(x.shape, x.dtype),
    )(seq_len_arr, x, emb_table, w2d)
